# baseline (speedup 1.0000x reference)
import jax
import jax.numpy as jnp
from jax.experimental import pallas as pl
from jax.experimental.pallas import tpu as pltpu

B, N = 128, 8192
CB = 1024
NBLK = N // CB
G = 128
NG = CB // G


def _tc_body(x_ref, m_ref, u_ref, o_ref, carry_ref):
    i = pl.program_id(0)

    @pl.when(i == 0)
    def _():
        carry_ref[...] = jnp.zeros_like(carry_ref)

    masked = x_ref[...] * m_ref[...].astype(jnp.float32)
    u = u_ref[...]
    off = carry_ref[...]
    for g in range(NG):
        s = jnp.dot(masked[:, g * G:(g + 1) * G], u,
                    preferred_element_type=jnp.float32)
        o_ref[:, g * G:(g + 1) * G] = s + off
        off = off + jnp.broadcast_to(s[:, G - 1:G], (B, G))
    carry_ref[...] = off


def kernel(x, mask):
    u = jnp.triu(jnp.ones((G, G), jnp.float32))
    return pl.pallas_call(
        _tc_body,
        grid=(NBLK,),
        in_specs=[
            pl.BlockSpec((B, CB), lambda i: (0, i)),
            pl.BlockSpec((B, CB), lambda i: (0, i)),
            pl.BlockSpec((G, G), lambda i: (0, 0)),
        ],
        out_specs=pl.BlockSpec((B, CB), lambda i: (0, i)),
        out_shape=jax.ShapeDtypeStruct((B, N), jnp.float32),
        scratch_shapes=[pltpu.VMEM((B, G), jnp.float32)],
    )(x, mask, u)


# speedup vs baseline: 1.3744x; 1.3744x over previous
import jax
import jax.numpy as jnp
from jax.experimental import pallas as pl
from jax.experimental.pallas import tpu as pltpu

B, N = 128, 8192
CB = 4096
NBLK = N // CB
G = 128
NG = CB // G


def _tc_body(x_ref, m_ref, u_ref, o_ref, carry_ref):
    i = pl.program_id(0)

    @pl.when(i == 0)
    def _():
        carry_ref[...] = jnp.zeros_like(carry_ref)

    masked = x_ref[...] * m_ref[...].astype(jnp.float32)
    u = u_ref[...]
    off = carry_ref[...]
    for g in range(NG):
        s = jnp.dot(masked[:, g * G:(g + 1) * G], u,
                    preferred_element_type=jnp.float32)
        o_ref[:, g * G:(g + 1) * G] = s + off
        off = off + jnp.broadcast_to(s[:, G - 1:G], (B, G))
    carry_ref[...] = off


def kernel(x, mask):
    u = jnp.triu(jnp.ones((G, G), jnp.float32))
    return pl.pallas_call(
        _tc_body,
        grid=(NBLK,),
        in_specs=[
            pl.BlockSpec((B, CB), lambda i: (0, i)),
            pl.BlockSpec((B, CB), lambda i: (0, i)),
            pl.BlockSpec((G, G), lambda i: (0, 0)),
        ],
        out_specs=pl.BlockSpec((B, CB), lambda i: (0, i)),
        out_shape=jax.ShapeDtypeStruct((B, N), jnp.float32),
        scratch_shapes=[pltpu.VMEM((B, G), jnp.float32)],
    )(x, mask, u)
